# baseline (speedup 1.0000x reference)
import functools

import jax
import jax.numpy as jnp
from jax import lax
from jax.experimental import pallas as pl
from jax.experimental.pallas import tpu as pltpu
from jax.experimental.pallas import tpu_sc as plsc

BATCH = 4
SEQ = 2048
DIM = 1024
TAB = 512
LANES = 16

NUM_CORES = 2
NUM_SUBCORES = 16
NW = NUM_CORES * NUM_SUBCORES
CHUNK = 16
NBUF = 4

SC_SEQ = 0

S_BLK = 512
B_BLK = 2


def _sc_add(embeddings, positions, pos_table, seq_len):
    seq_per_w = seq_len // NW
    rounds = seq_per_w // CHUNK
    mesh = plsc.VectorSubcoreMesh(
        core_axis_name="c", subcore_axis_name="s",
        num_cores=NUM_CORES, num_subcores=NUM_SUBCORES,
    )

    @functools.partial(
        pl.kernel,
        out_type=jax.ShapeDtypeStruct((BATCH, seq_len, DIM), jnp.float32),
        mesh=mesh,
        scratch_types=[
            pltpu.VMEM((seq_per_w,), jnp.int32),
            pltpu.VMEM((CHUNK, DIM), jnp.float32),
            pltpu.VMEM((CHUNK, DIM), jnp.float32),
            pltpu.VMEM((NBUF, CHUNK, DIM), jnp.float32),
            pltpu.SemaphoreType.DMA,
            pltpu.SemaphoreType.DMA,
            pltpu.SemaphoreType.DMA,
            pltpu.SemaphoreType.DMA,
        ],
    )
    def k(emb_hbm, pos_hbm, table_hbm, out_hbm,
          idx_v, rows0, rows1, ering, si, so, sg0, sg1):
        wid = lax.axis_index("s") * NUM_CORES + lax.axis_index("c")
        base = wid * seq_per_w
        pltpu.sync_copy(pos_hbm.at[pl.ds(base, seq_per_w)], idx_v)

        rows = (rows0, rows1)
        sg = (sg0, sg1)
        tasks = [(r, b) for r in range(rounds) for b in range(BATCH)]
        T = len(tasks)

        def gather(r):
            return pltpu.async_copy(
                table_hbm.at[idx_v.at[pl.ds(r * CHUNK, CHUNK)]],
                rows[r % 2], sg[r % 2],
            )

        def copy_in(t):
            r, b = tasks[t]
            return pltpu.async_copy(
                emb_hbm.at[b, pl.ds(base + r * CHUNK, CHUNK)],
                ering.at[t % NBUF], si,
            )

        g = [None] * rounds
        g[0] = gather(0)
        if rounds > 1:
            g[1] = gather(1)
        in_copies = [None] * T
        out_copies = [None] * T
        in_copies[0] = copy_in(0)
        if T > 1:
            in_copies[1] = copy_in(1)

        for t, (r, b) in enumerate(tasks):
            p = t % NBUF
            if b == 0:
                g[r].wait()
            in_copies[t].wait()

            @pl.loop(0, CHUNK)
            def _(row):
                @plsc.parallel_loop(0, DIM, LANES, unroll=8)
                def _(c):
                    x = rows[r % 2][row, pl.ds(c, LANES)]
                    plsc.addupdate(ering.at[p, row, pl.ds(c, LANES)], x)

            if b == BATCH - 1 and r + 2 < rounds:
                g[r + 2] = gather(r + 2)

            out_copies[t] = pltpu.async_copy(
                ering.at[p], out_hbm.at[b, pl.ds(base + r * CHUNK, CHUNK)], so,
            )
            nxt = t + 2
            if nxt < T:
                if t >= 2:
                    out_copies[t - 2].wait()
                in_copies[nxt] = copy_in(nxt)

        for t in range(max(0, T - NBUF), T):
            out_copies[t].wait()

    return k(embeddings, positions, pos_table)


def _tc_add(embeddings, positions, pos_table, seq_len):
    n_blk = seq_len // S_BLK
    pos3 = positions.reshape(n_blk, 1, S_BLK)

    def body(pos_ref, tab_ref, emb_ref, out_ref):
        pos = pos_ref[0, 0, :]
        onehot = (
            pos[:, None]
            == lax.broadcasted_iota(jnp.int32, (S_BLK, TAB), 1)
        ).astype(jnp.bfloat16)
        tab = tab_ref[...]
        tab_hi = tab.astype(jnp.bfloat16)
        tab_lo = (tab - tab_hi.astype(jnp.float32)).astype(jnp.bfloat16)
        dn = (((1,), (0,)), ((), ()))
        rows = lax.dot_general(
            onehot, tab_hi, dn, preferred_element_type=jnp.float32,
        ) + lax.dot_general(
            onehot, tab_lo, dn, preferred_element_type=jnp.float32,
        )
        out_ref[...] = emb_ref[...] + rows[None, :, :]

    return pl.pallas_call(
        body,
        grid=(n_blk, BATCH // B_BLK),
        in_specs=[
            pl.BlockSpec((1, 1, S_BLK), lambda i, j: (i, 0, 0)),
            pl.BlockSpec((TAB, DIM), lambda i, j: (0, 0)),
            pl.BlockSpec((B_BLK, S_BLK, DIM), lambda i, j: (j, i, 0)),
        ],
        out_specs=pl.BlockSpec((B_BLK, S_BLK, DIM), lambda i, j: (j, i, 0)),
        out_shape=jax.ShapeDtypeStruct((BATCH, seq_len, DIM), jnp.float32),
    )(pos3, pos_table, embeddings)


def kernel(embeddings, positions, pos_table):
    if SC_SEQ == 0:
        return _tc_add(embeddings, positions, pos_table, SEQ)
    if SC_SEQ == SEQ:
        return _sc_add(embeddings, positions, pos_table, SEQ)
    sc_out = _sc_add(
        embeddings[:, :SC_SEQ], positions[:SC_SEQ], pos_table, SC_SEQ)
    tc_out = _tc_add(
        embeddings[:, SC_SEQ:], positions[SC_SEQ:], pos_table, SEQ - SC_SEQ)
    full = jnp.concatenate([sc_out, tc_out], axis=1)
    return full


# speedup vs baseline: 1.0941x; 1.0941x over previous
import jax
import jax.numpy as jnp
from jax import lax
from jax.experimental import pallas as pl

BATCH = 4
SEQ = 2048
DIM = 1024
TAB = 512

S_BLK = 512


def kernel(embeddings, positions, pos_table):
    n_blk = SEQ // S_BLK
    pos3 = positions.reshape(n_blk, 1, S_BLK)

    def body(pos_ref, tab_ref, emb_ref, out_ref):
        pos = pos_ref[0, 0, :]
        onehot = (
            pos[:, None]
            == lax.broadcasted_iota(jnp.int32, (S_BLK, TAB), 1)
        ).astype(jnp.bfloat16)
        tab = tab_ref[...]
        tab_hi = tab.astype(jnp.bfloat16)
        tab_lo = (tab - tab_hi.astype(jnp.float32)).astype(jnp.bfloat16)
        dn = (((1,), (0,)), ((), ()))
        rows = lax.dot_general(
            onehot, tab_hi, dn, preferred_element_type=jnp.float32,
        ) + lax.dot_general(
            onehot, tab_lo, dn, preferred_element_type=jnp.float32,
        )
        out_ref[...] = emb_ref[...] + rows[None, :, :]

    return pl.pallas_call(
        body,
        grid=(n_blk,),
        in_specs=[
            pl.BlockSpec((1, 1, S_BLK), lambda i: (i, 0, 0)),
            pl.BlockSpec((TAB, DIM), lambda i: (0, 0)),
            pl.BlockSpec((BATCH, S_BLK, DIM), lambda i: (0, i, 0)),
        ],
        out_specs=pl.BlockSpec((BATCH, S_BLK, DIM), lambda i: (0, i, 0)),
        out_shape=jax.ShapeDtypeStruct((BATCH, SEQ, DIM), jnp.float32),
    )(pos3, pos_table, embeddings)
